# baseline (speedup 1.0000x reference)
import functools

import jax
import jax.numpy as jnp
from jax import lax
from jax.experimental import pallas as pl
from jax.experimental.pallas import tpu as pltpu
from jax.experimental.pallas import tpu_sc as plsc

N_NODES = 10000
N_EDGES = 320000
D_IN = 128
D_OUT = 128

DA = 160
NC = 2
NS = 16
NW = NC * NS
CHUNK = 80
N_CHUNKS = 128
GRP = 8
N_GRPS = N_CHUNKS // GRP
EDGES_PER_TILE = CHUNK * N_CHUNKS
E_PAD = EDGES_PER_TILE * NW
N_PAD = 10112
ROWS_PER_TILE = N_PAD // NS
PAD_DST = N_NODES


def _sc_scatter(xa, idx5, zeros):
    mesh = plsc.VectorSubcoreMesh(
        core_axis_name="c", subcore_axis_name="s", num_cores=NC, num_subcores=NS
    )

    @functools.partial(
        pl.kernel,
        out_type=jax.ShapeDtypeStruct((NC, N_PAD, DA), jnp.bfloat16),
        mesh=mesh,
        scratch_types=[
            pltpu.VMEM((2, GRP, CHUNK), jnp.int32),
            pltpu.VMEM((2, GRP, CHUNK), jnp.int32),
            pltpu.VMEM((CHUNK, DA), jnp.bfloat16),
            pltpu.VMEM((CHUNK, DA), jnp.bfloat16),
            pltpu.VMEM((CHUNK, DA), jnp.bfloat16),
            pltpu.VMEM((CHUNK, DA), jnp.bfloat16),
            pltpu.VMEM_SHARED((N_PAD, DA), jnp.bfloat16),
            pltpu.VMEM_SHARED((N_PAD, DA), jnp.bfloat16),
            pltpu.SemaphoreType.DMA,
            pltpu.SemaphoreType.DMA,
            pltpu.SemaphoreType.DMA,
            pltpu.SemaphoreType.DMA,
            pltpu.SemaphoreType.DMA,
        ],
        compiler_params=pltpu.CompilerParams(use_tc_tiling_on_sc=False),
    )
    def k(xa_hbm, idx_hbm, zeros_hbm, out_hbm,
          grp_a, grp_b, rows_a, rows_b, rows_c, rows_d,
          acc_sh, xa_sh, gsem_a, gsem_b, gsem_c, gsem_d, isem):
        c = lax.axis_index("c")
        s = lax.axis_index("s")
        w = c * NS + s
        row0 = s * ROWS_PER_TILE
        grp = (grp_a, grp_b)
        rows = (rows_a, rows_b, rows_c, rows_d)
        gsem = (gsem_a, gsem_b, gsem_c, gsem_d)

        def gather_start(p, j, x):
            pltpu.async_copy(xa_sh.at[grp[p].at[0, j]], rows[x], gsem[x])

        def gather_wait(p, j, x):
            pltpu.make_async_copy(xa_sh.at[grp[p].at[0, j]], rows[x], gsem[x]).wait()

        def scatter(p, j, x):
            pltpu.sync_copy(rows[x], acc_sh.at[grp[p].at[1, j]], add=True)

        def idx_start(g, p):
            pltpu.async_copy(idx_hbm.at[w, g], grp[p], isem)

        def idx_wait(g, p):
            pltpu.make_async_copy(idx_hbm.at[w, g], grp[p], isem).wait()

        pltpu.sync_copy(idx_hbm.at[w, 0], grp[0])
        pltpu.async_copy(idx_hbm.at[w, 1], grp[1], isem)
        pltpu.sync_copy(
            xa_hbm.at[pl.ds(row0, ROWS_PER_TILE)],
            xa_sh.at[pl.ds(row0, ROWS_PER_TILE)],
        )
        pltpu.sync_copy(
            zeros_hbm.at[pl.ds(row0, ROWS_PER_TILE)],
            acc_sh.at[pl.ds(row0, ROWS_PER_TILE)],
        )
        plsc.subcore_barrier()
        gather_start(0, 0, 0)
        gather_start(0, 1, 1)
        gather_start(0, 2, 2)

        def chunk_step(g, j, p, prefetch, last_grp):
            x = j % 4
            gather_wait(p, j, x)
            if j == 4 and not last_grp:
                idx_wait(g + 1, 1 - p)
            if not (last_grp and j >= GRP - 3):
                nj, np_ = (j + 3, p) if j + 3 < GRP else (j + 3 - GRP, 1 - p)
                gather_start(np_, nj, (j + 3) % 4)
            scatter(p, j, x)
            if j == 0 and prefetch:
                idx_start(g + 1, 1 - p)

        def group_pair(g0, first, last):
            for j in range(GRP):
                chunk_step(g0, j, 0, not first, False)
            for j in range(GRP):
                chunk_step(g0 + 1, j, 1, not last, last)

        group_pair(0, True, False)

        def body(t, carry):
            group_pair(t * 2, False, False)
            return carry

        lax.fori_loop(1, N_GRPS // 2 - 1, body, 0)
        group_pair(N_GRPS - 2, False, True)

        plsc.subcore_barrier()
        pltpu.sync_copy(
            acc_sh.at[pl.ds(row0, ROWS_PER_TILE)],
            out_hbm.at[c, pl.ds(row0, ROWS_PER_TILE)],
        )

    return k(xa, idx5, zeros)


def _tc_body(x_ref, p_ref, ws_ref, wn_ref, b_ref, o_ref):
    p = sum(p_ref[h].astype(jnp.float32) for h in range(NC))
    deg = p[:, D_IN : D_IN + 1]
    mean = p[:, :D_IN] / jnp.maximum(deg, 1.0)
    o_ref[...] = (
        jnp.dot(x_ref[...], ws_ref[...], preferred_element_type=jnp.float32)
        + jnp.dot(mean, wn_ref[...], preferred_element_type=jnp.float32)
        + b_ref[...]
    )


def _tc_dense(x, parts, W_self, W_neigh, b2):
    bm = 1000
    grid = N_NODES // bm
    return pl.pallas_call(
        _tc_body,
        out_shape=jax.ShapeDtypeStruct((N_NODES, D_OUT), jnp.float32),
        grid=(grid,),
        in_specs=[
            pl.BlockSpec((bm, D_IN), lambda i: (i, 0)),
            pl.BlockSpec((NC, bm, DA), lambda i: (0, i, 0)),
            pl.BlockSpec((D_IN, D_OUT), lambda i: (0, 0)),
            pl.BlockSpec((D_IN, D_OUT), lambda i: (0, 0)),
            pl.BlockSpec((1, D_OUT), lambda i: (0, 0)),
        ],
        out_specs=pl.BlockSpec((bm, D_OUT), lambda i: (i, 0)),
    )(x, parts, W_self, W_neigh, b2)


def kernel(x, edge_index, W_self, W_neigh, b):
    src = edge_index[0].astype(jnp.int32)
    dst = edge_index[1].astype(jnp.int32)
    fill_per_tile = EDGES_PER_TILE - N_EDGES // NW
    fidx = jnp.arange(NW * fill_per_tile, dtype=jnp.int32).reshape(NW, fill_per_tile)
    fill_src = (fidx * 131) % N_NODES
    fill_dst = PAD_DST + fidx % (N_PAD - N_NODES)
    src_t = jnp.concatenate([src.reshape(NW, -1), fill_src], axis=1)
    dst_t = jnp.concatenate([dst.reshape(NW, -1), fill_dst], axis=1)
    idx5 = jnp.stack(
        [
            src_t.reshape(NW, N_GRPS, GRP, CHUNK),
            dst_t.reshape(NW, N_GRPS, GRP, CHUNK),
        ],
        axis=2,
    )
    xa = jnp.concatenate(
        [
            x.astype(jnp.bfloat16),
            jnp.ones((N_NODES, 1), jnp.bfloat16),
            jnp.zeros((N_NODES, DA - D_IN - 1), jnp.bfloat16),
        ],
        axis=1,
    )
    xa = jnp.concatenate([xa, jnp.zeros((N_PAD - N_NODES, DA), jnp.bfloat16)])
    zeros = jnp.zeros((N_PAD, DA), jnp.bfloat16)
    parts = _sc_scatter(xa, idx5, zeros)
    return _tc_dense(x, parts, W_self, W_neigh, b.reshape(1, D_OUT))


# speedup vs baseline: 1.0645x; 1.0645x over previous
import functools

import jax
import jax.numpy as jnp
from jax import lax
from jax.experimental import pallas as pl
from jax.experimental.pallas import tpu as pltpu
from jax.experimental.pallas import tpu_sc as plsc

N_NODES = 10000
N_EDGES = 320000
D_IN = 128
D_OUT = 128

DA = 160
NC = 2
NS = 16
NW = NC * NS
CHUNK = 128
N_CHUNKS = 80
GRP = 8
N_GRPS = N_CHUNKS // GRP
EDGES_PER_TILE = CHUNK * N_CHUNKS
E_PAD = EDGES_PER_TILE * NW
N_PAD = 10112
ROWS_PER_TILE = N_PAD // NS
PAD_DST = N_NODES


def _sc_scatter(xa, idx5, zeros):
    mesh = plsc.VectorSubcoreMesh(
        core_axis_name="c", subcore_axis_name="s", num_cores=NC, num_subcores=NS
    )

    @functools.partial(
        pl.kernel,
        out_type=jax.ShapeDtypeStruct((NC, N_PAD, DA), jnp.bfloat16),
        mesh=mesh,
        scratch_types=[
            pltpu.VMEM((2, GRP, CHUNK), jnp.int32),
            pltpu.VMEM((2, GRP, CHUNK), jnp.int32),
            pltpu.VMEM((CHUNK, DA), jnp.bfloat16),
            pltpu.VMEM((CHUNK, DA), jnp.bfloat16),
            pltpu.VMEM_SHARED((N_PAD, DA), jnp.bfloat16),
            pltpu.VMEM_SHARED((N_PAD, DA), jnp.bfloat16),
            pltpu.SemaphoreType.DMA,
            pltpu.SemaphoreType.DMA,
            pltpu.SemaphoreType.DMA,
        ],
        compiler_params=pltpu.CompilerParams(use_tc_tiling_on_sc=False),
    )
    def k(xa_hbm, idx_hbm, zeros_hbm, out_hbm,
          grp_a, grp_b, rows_a, rows_b,
          acc_sh, xa_sh, gsem_a, gsem_b, isem):
        c = lax.axis_index("c")
        s = lax.axis_index("s")
        w = c * NS + s
        row0 = s * ROWS_PER_TILE
        grp = (grp_a, grp_b)
        rows = (rows_a, rows_b)
        gsem = (gsem_a, gsem_b)

        def gather_start(p, j, x):
            pltpu.async_copy(xa_sh.at[grp[p].at[0, j]], rows[x], gsem[x])

        def gather_wait(p, j, x):
            pltpu.make_async_copy(xa_sh.at[grp[p].at[0, j]], rows[x], gsem[x]).wait()

        def scatter(p, j, x):
            pltpu.sync_copy(rows[x], acc_sh.at[grp[p].at[1, j]], add=True)

        def idx_start(g, p):
            pltpu.async_copy(idx_hbm.at[w, g], grp[p], isem)

        def idx_wait(g, p):
            pltpu.make_async_copy(idx_hbm.at[w, g], grp[p], isem).wait()

        pltpu.sync_copy(idx_hbm.at[w, 0], grp[0])
        pltpu.sync_copy(
            xa_hbm.at[pl.ds(row0, ROWS_PER_TILE)],
            xa_sh.at[pl.ds(row0, ROWS_PER_TILE)],
        )
        pltpu.sync_copy(
            zeros_hbm.at[pl.ds(row0, ROWS_PER_TILE)],
            acc_sh.at[pl.ds(row0, ROWS_PER_TILE)],
        )
        plsc.subcore_barrier()
        gather_start(0, 0, 0)

        def chunk_step(g, j, p, first, last, prefetch):
            x = j % 2
            o = 1 - x
            gather_wait(p, j, x)
            if not last:
                nj, np_ = (j + 1, p) if j + 1 < GRP else (0, 1 - p)
                if j + 1 == GRP:
                    idx_wait(g + 1, np_)
                gather_start(np_, nj, o)
            scatter(p, j, x)
            if j == 0 and prefetch:
                idx_start(g + 1, 1 - p)

        def group_pair(g0, first, last):
            for j in range(GRP):
                chunk_step(g0, j, 0, first and j == 0, False, j == 0)
            for j in range(GRP):
                chunk_step(g0 + 1, j, 1, False, last and j == GRP - 1,
                           j == 0 and not last)

        group_pair(0, True, False)

        def body(t, carry):
            group_pair(t * 2, False, False)
            return carry

        lax.fori_loop(1, N_GRPS // 2 - 1, body, 0)
        group_pair(N_GRPS - 2, False, True)

        plsc.subcore_barrier()
        pltpu.sync_copy(
            acc_sh.at[pl.ds(row0, ROWS_PER_TILE)],
            out_hbm.at[c, pl.ds(row0, ROWS_PER_TILE)],
        )

    return k(xa, idx5, zeros)


def _tc_body(x_ref, p_ref, ws_ref, wn_ref, b_ref, o_ref):
    p = sum(p_ref[h].astype(jnp.float32) for h in range(NC))
    deg = p[:, D_IN : D_IN + 1]
    mean = p[:, :D_IN] / jnp.maximum(deg, 1.0)
    o_ref[...] = (
        jnp.dot(x_ref[...], ws_ref[...], preferred_element_type=jnp.float32)
        + jnp.dot(mean, wn_ref[...], preferred_element_type=jnp.float32)
        + b_ref[...]
    )


def _tc_dense(x, parts, W_self, W_neigh, b2):
    bm = 1000
    grid = N_NODES // bm
    return pl.pallas_call(
        _tc_body,
        out_shape=jax.ShapeDtypeStruct((N_NODES, D_OUT), jnp.float32),
        grid=(grid,),
        in_specs=[
            pl.BlockSpec((bm, D_IN), lambda i: (i, 0)),
            pl.BlockSpec((NC, bm, DA), lambda i: (0, i, 0)),
            pl.BlockSpec((D_IN, D_OUT), lambda i: (0, 0)),
            pl.BlockSpec((D_IN, D_OUT), lambda i: (0, 0)),
            pl.BlockSpec((1, D_OUT), lambda i: (0, 0)),
        ],
        out_specs=pl.BlockSpec((bm, D_OUT), lambda i: (i, 0)),
    )(x, parts, W_self, W_neigh, b2)


def kernel(x, edge_index, W_self, W_neigh, b):
    src = edge_index[0].astype(jnp.int32)
    dst = edge_index[1].astype(jnp.int32)
    fill_per_tile = EDGES_PER_TILE - N_EDGES // NW
    fidx = jnp.arange(NW * fill_per_tile, dtype=jnp.int32).reshape(NW, fill_per_tile)
    fill_src = (fidx * 131) % N_NODES
    fill_dst = PAD_DST + fidx % (N_PAD - N_NODES)
    src_t = jnp.concatenate([src.reshape(NW, -1), fill_src], axis=1)
    dst_t = jnp.concatenate([dst.reshape(NW, -1), fill_dst], axis=1)
    idx5 = jnp.stack(
        [
            src_t.reshape(NW, N_GRPS, GRP, CHUNK),
            dst_t.reshape(NW, N_GRPS, GRP, CHUNK),
        ],
        axis=2,
    )
    xa = jnp.concatenate(
        [
            x.astype(jnp.bfloat16),
            jnp.ones((N_NODES, 1), jnp.bfloat16),
            jnp.zeros((N_NODES, DA - D_IN - 1), jnp.bfloat16),
        ],
        axis=1,
    )
    xa = jnp.concatenate([xa, jnp.zeros((N_PAD - N_NODES, DA), jnp.bfloat16)])
    zeros = jnp.zeros((N_PAD, DA), jnp.bfloat16)
    parts = _sc_scatter(xa, idx5, zeros)
    return _tc_dense(x, parts, W_self, W_neigh, b.reshape(1, D_OUT))
